# baseline (speedup 1.0000x reference)
import jax
import jax.numpy as jnp
from jax import lax
from jax.experimental import pallas as pl
from jax.experimental.pallas import tpu as pltpu
from jax.experimental.pallas import tpu_sc as plsc

NC = 2
NS = 16
L = 16
NW = NC * NS
K = 500
KPAD = 512
ROWS_PER_W = 32
HW = 128 * 128
TOTW = ROWS_PER_W * KPAD
GCH = 512
NDMA = TOTW // GCH
CPD = GCH // L


def _sc_body(pred_hbm, packed_hbm, gt_hbm, out_hbm,
             idxf_v, m_v, ibuf, dbuf, gtbuf, out_v, gsem, ssem):
    cid = lax.axis_index("c")
    sid = lax.axis_index("s")
    wid = sid * NC + cid
    b = wid // 2
    c0 = (wid % 2) * ROWS_PER_W
    zf = jnp.zeros((L,), jnp.float32)

    pltpu.sync_copy(packed_hbm.at[b, 0], idxf_v)
    pltpu.sync_copy(packed_hbm.at[b, 1], m_v)
    gtc = pltpu.make_async_copy(gt_hbm.at[b, pl.ds(c0, ROWS_PER_W)], gtbuf,
                                ssem)
    gtc.start()

    base0 = (b * 64 + c0) * HW

    def build_row(j, _):
        base = base0 + j * HW
        o = j * KPAD

        def build(t, _):
            iv = idxf_v[pl.ds(t * L, L)].astype(jnp.int32)
            ibuf[pl.ds(o + t * L, L)] = iv + base
            return 0
        lax.fori_loop(0, KPAD // L, build, 0)

        def fire(r, _):
            pltpu.make_async_copy(
                pred_hbm.at[ibuf.at[pl.ds(o + r * GCH, GCH)]],
                dbuf.at[pl.ds(o + r * GCH, GCH)], gsem).start()
            return 0
        lax.fori_loop(0, KPAD // GCH, fire, 0)
        return 0
    lax.fori_loop(0, ROWS_PER_W, build_row, 0)

    gtc.wait()

    def drain(r, acc):
        pltpu.make_async_copy(
            pred_hbm.at[ibuf.at[pl.ds(0, GCH)]],
            dbuf.at[pl.ds(0, GCH)], gsem).wait()
        def inner(t, a):
            ct = r * CPD + t
            g = dbuf[pl.ds(ct * L, L)]
            m = m_v[pl.ds(t * L, L)]
            gv = gtbuf[r, pl.ds(t * L, L)]
            return a + jnp.abs(g * m - gv * m)
        return lax.fori_loop(0, CPD, inner, acc)
    acc = lax.fori_loop(0, NDMA, drain, zf)

    def m_step(t, a):
        return a + m_v[pl.ds(t * L, L)]
    acc_m = lax.fori_loop(0, KPAD // L, m_step, zf) * float(ROWS_PER_W)

    out_v[pl.ds(0, L)] = acc
    out_v[pl.ds(L, L)] = acc_m
    pltpu.sync_copy(out_v, out_hbm.at[wid])


@jax.jit
def _run(pred_flat, packed, gt):
    mesh = plsc.VectorSubcoreMesh(core_axis_name="c", subcore_axis_name="s",
                                  num_cores=NC, num_subcores=NS)
    f = pl.kernel(
        _sc_body,
        out_type=jax.ShapeDtypeStruct((NW, 2 * L), jnp.float32),
        mesh=mesh,
        scratch_types=[
            pltpu.VMEM((KPAD,), jnp.float32),
            pltpu.VMEM((KPAD,), jnp.float32),
            pltpu.VMEM((TOTW,), jnp.int32),
            pltpu.VMEM((TOTW,), jnp.float32),
            pltpu.VMEM((ROWS_PER_W, KPAD), jnp.float32),
            pltpu.VMEM((2 * L,), jnp.float32),
            pltpu.SemaphoreType.DMA,
            pltpu.SemaphoreType.DMA,
        ],
        compiler_params=pltpu.CompilerParams(needs_layout_passes=False),
    )
    out = f(pred_flat, packed, gt)
    return out[:, :L].sum() / (out[:, L:].sum() + 0.0001)


def kernel(pred, inds, ind_mask, gt):
    k = inds.shape[1]
    pad = KPAD - k
    indsf = inds.astype(jnp.float32)
    packed = jnp.pad(jnp.stack([indsf, ind_mask], axis=1),
                     ((0, 0), (0, 0), (0, pad)))
    gt_p = jnp.pad(gt, ((0, 0), (0, 0), (0, pad)))
    return _run(pred.reshape(-1), packed, gt_p)


# speedup vs baseline: 1.0391x; 1.0391x over previous
import jax
import jax.numpy as jnp
from jax import lax
from jax.experimental import pallas as pl
from jax.experimental.pallas import tpu as pltpu
from jax.experimental.pallas import tpu_sc as plsc

NC = 2
NS = 16
L = 16
NW = NC * NS
K = 500
KPAD = 512
ROWS_PER_W = 32
HW = 128 * 128
TOTW = ROWS_PER_W * KPAD
GCH = 128
NDMA = TOTW // GCH
CPD = GCH // L


def _sc_body(pred_hbm, packed_hbm, gt_hbm, out_hbm,
             idxf_v, m_v, ibuf, dbuf, gtbuf, out_v, gsem, ssem):
    cid = lax.axis_index("c")
    sid = lax.axis_index("s")
    wid = sid * NC + cid
    b = wid // 2
    c0 = (wid % 2) * ROWS_PER_W
    zf = jnp.zeros((L,), jnp.float32)

    pltpu.sync_copy(packed_hbm.at[b, 0], idxf_v)
    pltpu.sync_copy(packed_hbm.at[b, 1], m_v)
    gtc = pltpu.make_async_copy(gt_hbm.at[b, pl.ds(c0, ROWS_PER_W)], gtbuf,
                                ssem)
    gtc.start()

    base0 = (b * 64 + c0) * HW

    def build_row(j, _):
        base = base0 + j * HW
        o = j * KPAD

        def build(t, _):
            iv = idxf_v[pl.ds(t * L, L)].astype(jnp.int32)
            ibuf[pl.ds(o + t * L, L)] = iv + base
            return 0
        lax.fori_loop(0, KPAD // L, build, 0)

        def fire(r, _):
            pltpu.make_async_copy(
                pred_hbm.at[ibuf.at[pl.ds(o + r * GCH, GCH)]],
                dbuf.at[pl.ds(o + r * GCH, GCH)], gsem).start()
            return 0
        lax.fori_loop(0, KPAD // GCH, fire, 0)
        return 0
    lax.fori_loop(0, ROWS_PER_W, build_row, 0)

    gtc.wait()

    def drain(r, acc):
        pltpu.make_async_copy(
            pred_hbm.at[ibuf.at[pl.ds(0, GCH)]],
            dbuf.at[pl.ds(0, GCH)], gsem).wait()
        j = r // (KPAD // GCH)
        tbase = lax.rem(r, KPAD // GCH) * CPD

        def inner(t, a):
            g = dbuf[pl.ds((r * CPD + t) * L, L)]
            m = m_v[pl.ds((tbase + t) * L, L)]
            gv = gtbuf[j, pl.ds((tbase + t) * L, L)]
            return a + jnp.abs(g * m - gv * m)
        return lax.fori_loop(0, CPD, inner, acc)
    acc = lax.fori_loop(0, NDMA, drain, zf)

    def m_step(t, a):
        return a + m_v[pl.ds(t * L, L)]
    acc_m = lax.fori_loop(0, KPAD // L, m_step, zf) * float(ROWS_PER_W)

    out_v[pl.ds(0, L)] = acc
    out_v[pl.ds(L, L)] = acc_m
    pltpu.sync_copy(out_v, out_hbm.at[wid])


@jax.jit
def _run(pred_flat, packed, gt):
    mesh = plsc.VectorSubcoreMesh(core_axis_name="c", subcore_axis_name="s",
                                  num_cores=NC, num_subcores=NS)
    f = pl.kernel(
        _sc_body,
        out_type=jax.ShapeDtypeStruct((NW, 2 * L), jnp.float32),
        mesh=mesh,
        scratch_types=[
            pltpu.VMEM((KPAD,), jnp.float32),
            pltpu.VMEM((KPAD,), jnp.float32),
            pltpu.VMEM((TOTW,), jnp.int32),
            pltpu.VMEM((TOTW,), jnp.float32),
            pltpu.VMEM((ROWS_PER_W, KPAD), jnp.float32),
            pltpu.VMEM((2 * L,), jnp.float32),
            pltpu.SemaphoreType.DMA,
            pltpu.SemaphoreType.DMA,
        ],
        compiler_params=pltpu.CompilerParams(needs_layout_passes=False),
    )
    out = f(pred_flat, packed, gt)
    return out[:, :L].sum() / (out[:, L:].sum() + 0.0001)


def kernel(pred, inds, ind_mask, gt):
    k = inds.shape[1]
    pad = KPAD - k
    indsf = inds.astype(jnp.float32)
    packed = jnp.pad(jnp.stack([indsf, ind_mask], axis=1),
                     ((0, 0), (0, 0), (0, pad)))
    gt_p = jnp.pad(gt, ((0, 0), (0, 0), (0, pad)))
    return _run(pred.reshape(-1), packed, gt_p)


# speedup vs baseline: 1.1051x; 1.0635x over previous
import jax
import jax.numpy as jnp
from jax import lax
from jax.experimental import pallas as pl
from jax.experimental.pallas import tpu as pltpu
from jax.experimental.pallas import tpu_sc as plsc

NC = 2
NS = 16
L = 16
NW = NC * NS
K = 500
KPAD = 512
ROWS_PER_W = 32
HW = 128 * 128
TOTW = ROWS_PER_W * KPAD
GCH = 128
NDMA = TOTW // GCH
CPD = GCH // L


def _sc_body(pred_hbm, packed_hbm, gt_hbm, out_hbm,
             idxf_v, m_v, ibuf, dbuf, gtbuf, out_v, gsem, ssem):
    cid = lax.axis_index("c")
    sid = lax.axis_index("s")
    wid = sid * NC + cid
    b = wid // 2
    c0 = (wid % 2) * ROWS_PER_W
    zf = jnp.zeros((L,), jnp.float32)

    pltpu.sync_copy(packed_hbm.at[b, 0], idxf_v)
    pltpu.sync_copy(packed_hbm.at[b, 1], m_v)
    gtc = pltpu.make_async_copy(gt_hbm.at[b, pl.ds(c0, ROWS_PER_W)], gtbuf,
                                ssem)
    gtc.start()

    base0 = (b * 64 + c0) * HW

    def build_row(j, _):
        base = base0 + j * HW
        o = j * KPAD

        def build(t, _):
            iv = idxf_v[pl.ds(t * L, L)].astype(jnp.int32)
            ibuf[pl.ds(o + t * L, L)] = iv + base
            return 0
        lax.fori_loop(0, KPAD // L, build, 0)

        def fire(r, _):
            pltpu.make_async_copy(
                pred_hbm.at[ibuf.at[pl.ds(o + r * GCH, GCH)]],
                dbuf.at[pl.ds(o + r * GCH, GCH)], gsem).start()
            return 0
        lax.fori_loop(0, KPAD // GCH, fire, 0)
        return 0
    lax.fori_loop(0, ROWS_PER_W, build_row, 0)

    gtc.wait()

    def drain(r, acc):
        pltpu.make_async_copy(
            pred_hbm.at[ibuf.at[pl.ds(0, GCH)]],
            dbuf.at[pl.ds(0, GCH)], gsem).wait()
        j = r // (KPAD // GCH)
        tbase = lax.rem(r, KPAD // GCH) * CPD

        def inner(t, a):
            tloc = tbase + t
            g = dbuf[pl.ds((r * CPD + t) * L, L)]
            m = m_v[pl.ds(tloc * L, L)]
            gv = lax.cond(
                tloc == KPAD // L - 1,
                lambda: plsc.load_gather(
                    gtbuf,
                    [jnp.full((L,), j, jnp.int32),
                     jnp.minimum(jax.lax.iota(jnp.int32, L) + (K - 4),
                                 K - 1)]),
                lambda: gtbuf[j, pl.ds(tloc * L, L)])
            return a + jnp.abs(g * m - gv * m)
        return lax.fori_loop(0, CPD, inner, acc)
    acc = lax.fori_loop(0, NDMA, drain, zf)

    def m_step(t, a):
        return a + m_v[pl.ds(t * L, L)]
    acc_m = lax.fori_loop(0, KPAD // L, m_step, zf) * float(ROWS_PER_W)

    out_v[pl.ds(0, L)] = acc
    out_v[pl.ds(L, L)] = acc_m
    pltpu.sync_copy(out_v, out_hbm.at[wid])


@jax.jit
def _run(pred_flat, packed, gt):
    mesh = plsc.VectorSubcoreMesh(core_axis_name="c", subcore_axis_name="s",
                                  num_cores=NC, num_subcores=NS)
    f = pl.kernel(
        _sc_body,
        out_type=jax.ShapeDtypeStruct((NW, 2 * L), jnp.float32),
        mesh=mesh,
        scratch_types=[
            pltpu.VMEM((KPAD,), jnp.float32),
            pltpu.VMEM((KPAD,), jnp.float32),
            pltpu.VMEM((TOTW,), jnp.int32),
            pltpu.VMEM((TOTW,), jnp.float32),
            pltpu.VMEM((ROWS_PER_W, K), jnp.float32),
            pltpu.VMEM((2 * L,), jnp.float32),
            pltpu.SemaphoreType.DMA,
            pltpu.SemaphoreType.DMA,
        ],
        compiler_params=pltpu.CompilerParams(needs_layout_passes=False),
    )
    out = f(pred_flat, packed, gt)
    return out[:, :L].sum() / (out[:, L:].sum() + 0.0001)


def kernel(pred, inds, ind_mask, gt):
    k = inds.shape[1]
    pad = KPAD - k
    indsf = inds.astype(jnp.float32)
    packed = jnp.pad(jnp.stack([indsf, ind_mask], axis=1),
                     ((0, 0), (0, 0), (0, pad)))
    return _run(pred.reshape(-1), packed, gt)
